# baseline (device time: 77859 ns/iter reference)
import jax
import jax.numpy as jnp
from jax import lax
from jax.experimental import pallas as pl
from jax.experimental.pallas import tpu as pltpu

N_DEV = 4
SQ = 1024
SKV = 1024
HQ = 8
DH = 128
D = HQ * DH
BLK = 64
QW = SQ // N_DEV
SCALE = 0.08838834764831843

ROWS = (768, 512, 0, 256)
PREFIX = (1024, 768, 256, 512)


def _body(x_ref, wq_ref, kv_ref, wo_ref, out_ref,
          comm_ref, ctx_ref, mine_ref,
          kv_ssems, kv_rsems, g_ssems, g_rsems):
    my = lax.axis_index("i")

    def mk_kv(h, tgt_idx, tgt):
        p = PREFIX[tgt]
        return pltpu.make_async_remote_copy(
            src_ref=comm_ref.at[h, :, pl.ds(0, p), :],
            dst_ref=comm_ref.at[h, :, pl.ds(0, p), :],
            send_sem=kv_ssems.at[tgt_idx, h],
            recv_sem=kv_rsems.at[h],
            device_id=(tgt,),
            device_id_type=pl.DeviceIdType.MESH,
        )

    to = {t: [mk_kv(h, i, t) for h in range(HQ)]
          for i, t in enumerate((1, 3))}
    to[2] = [mk_kv(h, 2, 2) for h in range(HQ)]

    @pl.when(my == 0)
    def _():
        comm_ref[...] = kv_ref[...]
        for h in range(HQ):
            to[1][h].start()
        for h in range(HQ):
            to[3][h].start()

    for d in (1, 2, 3):
        @pl.when(my == d)
        def _(d=d):
            comm_ref[:, 1, PREFIX[d]:, :] = jnp.zeros(
                (HQ, SKV - PREFIX[d], DH), jnp.bfloat16)

    start = jnp.where(my == 0, ROWS[0],
                      jnp.where(my == 1, ROWS[1],
                                jnp.where(my == 2, ROWS[2], ROWS[3])))

    xq = x_ref[0, pl.ds(start, QW), :].astype(jnp.bfloat16)
    wqb = wq_ref[...].astype(jnp.bfloat16)
    qq = jnp.dot(xq, wqb,
                 preferred_element_type=jnp.float32).astype(jnp.bfloat16)

    qb = (start + lax.broadcasted_iota(jnp.int32, (QW, SKV), 0)) // BLK
    kb = lax.broadcasted_iota(jnp.int32, (QW, SKV), 1) // BLK
    mask = kb <= qb

    for h in range(HQ):
        for d in (1, 2, 3):
            @pl.when(my == d)
            def _(h=h, d=d):
                to[d][h].wait_recv()

        @pl.when(my == 3)
        def _(h=h):
            to[2][h].start()

        k = comm_ref[h, 0]
        v = comm_ref[h, 1]
        qh = qq[:, h * DH:(h + 1) * DH]
        s = lax.dot_general(
            qh, k, (((1,), (1,)), ((), ())),
            preferred_element_type=jnp.float32,
        ) * SCALE
        w = jnp.where(mask, jnp.exp(s), 0.0)
        p = (w / jnp.sum(w, axis=1, keepdims=True)).astype(jnp.bfloat16)
        ctx = jnp.dot(p, v, preferred_element_type=jnp.float32)
        ctx_ref[:, h * DH:(h + 1) * DH] = ctx.astype(jnp.bfloat16)

    wob = wo_ref[...].astype(jnp.bfloat16)
    myout = jnp.dot(ctx_ref[...], wob, preferred_element_type=jnp.float32)
    mine_ref[...] = myout
    out_ref[pl.ds(start, QW), :] = myout

    gath = {}
    for o in range(N_DEV):
        others = [t for t in range(N_DEV) if t != o]
        gath[o] = [
            pltpu.make_async_remote_copy(
                src_ref=mine_ref,
                dst_ref=out_ref.at[pl.ds(ROWS[o], QW), :],
                send_sem=g_ssems.at[i],
                recv_sem=g_rsems.at[o],
                device_id=(t,),
                device_id_type=pl.DeviceIdType.MESH,
            )
            for i, t in enumerate(others)
        ]

    for o in range(N_DEV):
        @pl.when(my == o)
        def _(o=o):
            for gd in gath[o]:
                gd.start()

    @pl.when(my == 0)
    def _():
        for t in (1, 3):
            for h in range(HQ):
                to[t][h].wait_send()

    @pl.when(my == 3)
    def _():
        for h in range(HQ):
            to[2][h].wait_send()

    for o in range(N_DEV):
        @pl.when(my == o)
        def _(o=o):
            for gd in gath[o]:
                gd.wait_send()

        @pl.when(my != o)
        def _(o=o):
            gath[o][0].wait_recv()


def kernel(x, Wq, K_ext, V_ext, Wo):
    bf16 = jnp.bfloat16
    kvb = jnp.stack(
        [K_ext[0].astype(bf16).transpose(1, 0, 2),
         V_ext[0].astype(bf16).transpose(1, 0, 2)],
        axis=1,
    )

    out = pl.pallas_call(
        _body,
        out_shape=jax.ShapeDtypeStruct((SQ, D), jnp.float32),
        in_specs=[pl.BlockSpec(memory_space=pltpu.VMEM)] * 4,
        out_specs=pl.BlockSpec(memory_space=pltpu.VMEM),
        scratch_shapes=[
            pltpu.VMEM((HQ, 2, SKV, DH), bf16),
            pltpu.VMEM((QW, D), bf16),
            pltpu.VMEM((QW, D), jnp.float32),
            pltpu.SemaphoreType.DMA((3, HQ)),
            pltpu.SemaphoreType.DMA((HQ,)),
            pltpu.SemaphoreType.DMA((3,)),
            pltpu.SemaphoreType.DMA((N_DEV,)),
        ],
    )(x, Wq, kvb, Wo)

    return out.reshape(1, SQ, D)


# device time: 74430 ns/iter; 1.0461x vs baseline; 1.0461x over previous
import jax
import jax.numpy as jnp
from jax import lax
from jax.experimental import pallas as pl
from jax.experimental.pallas import tpu as pltpu

N_DEV = 4
SQ = 1024
SKV = 1024
HQ = 8
DH = 128
D = HQ * DH
BLK = 64
SCALE = 0.08838834764831843

ORDER_01 = (0, 1, 2, 3, 4, 5, 6, 7)
ORDER_3 = (4, 5, 6, 7, 0, 1, 2, 3)
ORDER_2 = (0, 4, 1, 5, 2, 6, 3, 7)


def _body(x_ref, wq_ref, kv_ref, wo_ref, out_ref,
          comm_ref, ctx_ref, send_sems, recv_sems):
    my = lax.axis_index("i")

    def mk(h, slot, tgt):
        return pltpu.make_async_remote_copy(
            src_ref=comm_ref.at[h],
            dst_ref=comm_ref.at[h],
            send_sem=send_sems.at[slot, h],
            recv_sem=recv_sems.at[h],
            device_id=(tgt,),
            device_id_type=pl.DeviceIdType.MESH,
        )

    to1 = [mk(h, 0, 1) for h in range(HQ)]
    to3 = [mk(h, 1, 3) for h in range(HQ)]
    rel = [mk(h, 0, 2) for h in range(HQ)]

    @pl.when(my == 0)
    def _():
        comm_ref[...] = kv_ref[...]
        for h in ORDER_01:
            to1[h].start()
        for h in ORDER_3:
            to3[h].start()

    xb = x_ref[0].astype(jnp.bfloat16)
    wqb = wq_ref[...].astype(jnp.bfloat16)
    q_all = jnp.dot(xb, wqb,
                    preferred_element_type=jnp.float32).astype(jnp.bfloat16)

    HALF = SQ // 2

    def blk_mask(q0, rows, cols):
        qb = (q0 + lax.broadcasted_iota(jnp.int32, (rows, cols), 0)) // BLK
        kb = lax.broadcasted_iota(jnp.int32, (rows, cols), 1) // BLK
        return kb <= qb

    mask_lo = blk_mask(0, HALF, HALF)
    mask_hi = blk_mask(HALF, HALF, SKV)

    def attend(qpart, k, v, mask):
        s = lax.dot_general(
            qpart, k, (((1,), (1,)), ((), ())),
            preferred_element_type=jnp.float32,
        ) * SCALE
        w = jnp.where(mask, jnp.exp(s), 0.0)
        p = (w / jnp.sum(w, axis=1, keepdims=True)).astype(jnp.bfloat16)
        return jnp.dot(p, v, preferred_element_type=jnp.float32)

    def compute_head(h):
        k = comm_ref[h, 0]
        v = comm_ref[h, 1]
        qh = q_all[:, h * DH:(h + 1) * DH]
        ctx_lo = attend(qh[:HALF], k[:HALF], v[:HALF], mask_lo)
        ctx_hi = attend(qh[HALF:], k, v, mask_hi)
        ctx_ref[:HALF, h * DH:(h + 1) * DH] = ctx_lo.astype(jnp.bfloat16)
        ctx_ref[HALF:, h * DH:(h + 1) * DH] = ctx_hi.astype(jnp.bfloat16)

    @pl.when(my < 2)
    def _():
        for h in ORDER_01:
            @pl.when(my == 1)
            def _(h=h):
                to1[h].wait_recv()
                if h < HQ // 2:
                    rel[h].start()
            compute_head(h)

    @pl.when(my == 3)
    def _():
        for h in ORDER_3:
            to3[h].wait_recv()
            if h >= HQ // 2:
                rel[h].start()
            compute_head(h)

    @pl.when(my == 2)
    def _():
        for h in ORDER_2:
            rel[h].wait_recv()
            compute_head(h)

    wob = wo_ref[...].astype(jnp.bfloat16)
    out_ref[...] = jnp.dot(ctx_ref[...], wob,
                           preferred_element_type=jnp.float32)

    @pl.when(my == 0)
    def _():
        for h in range(HQ):
            to1[h].wait_send()
            to3[h].wait_send()

    @pl.when(my == 1)
    def _():
        for h in range(HQ // 2):
            rel[h].wait_send()

    @pl.when(my == 3)
    def _():
        for h in range(HQ // 2, HQ):
            rel[h].wait_send()


def kernel(x, Wq, K_ext, V_ext, Wo):
    bf16 = jnp.bfloat16
    kvb = jnp.stack(
        [K_ext[0].astype(bf16).transpose(1, 0, 2),
         V_ext[0].astype(bf16).transpose(1, 0, 2)],
        axis=1,
    )

    out = pl.pallas_call(
        _body,
        out_shape=jax.ShapeDtypeStruct((SQ, D), jnp.float32),
        in_specs=[pl.BlockSpec(memory_space=pltpu.VMEM)] * 4,
        out_specs=pl.BlockSpec(memory_space=pltpu.VMEM),
        scratch_shapes=[
            pltpu.VMEM((HQ, 2, SKV, DH), bf16),
            pltpu.VMEM((SQ, D), bf16),
            pltpu.SemaphoreType.DMA((2, HQ)),
            pltpu.SemaphoreType.DMA((HQ,)),
        ],
    )(x, Wq, kvb, Wo)

    return out.reshape(1, SQ, D)
